# baseline (speedup 1.0000x reference)
import functools

import jax
import jax.numpy as jnp
from jax import lax
from jax.experimental import pallas as pl
from jax.experimental.pallas import tpu as pltpu
from jax.experimental.pallas import tpu_sc as plsc

CHUNK = 128
K = 8


@functools.partial(jax.jit, static_argnums=(2, 3))
def _gather_rows(weight, idx2, n_rows, d):
    info = plsc.get_sparse_core_info()
    nc, ns = info.num_cores, info.num_subcores
    nw = nc * ns
    rows_pw = n_rows // nw

    mesh = plsc.VectorSubcoreMesh(core_axis_name="c", subcore_axis_name="s")

    @functools.partial(
        pl.kernel,
        mesh=mesh,
        compiler_params=pltpu.CompilerParams(use_tc_tiling_on_sc=False),
        out_type=jax.ShapeDtypeStruct((n_rows, CHUNK, d), jnp.float32),
        scratch_types=[
            pltpu.VMEM((K, CHUNK), jnp.int32),
            pltpu.VMEM((K, CHUNK, d), jnp.float32),
            pltpu.SemaphoreType.DMA,
        ],
    )
    def k(table_hbm, idx_hbm, out_hbm, idx_v, rows_v, sem):
        wid = lax.axis_index("s") * nc + lax.axis_index("c")
        base = wid * rows_pw

        def body(i, carry):
            r = base + i * K
            pltpu.sync_copy(idx_hbm.at[pl.ds(r, K)], idx_v)
            copies = [
                pltpu.async_copy(table_hbm.at[idx_v.at[j]], rows_v.at[j], sem)
                for j in range(K)
            ]
            for c in copies:
                c.wait()
            pltpu.sync_copy(rows_v, out_hbm.at[pl.ds(r, K)])
            return carry

        lax.fori_loop(0, rows_pw // K, body, 0)

    return k(weight, idx2)


def kernel(token_ids, weight):
    b0, b1 = token_ids.shape
    v, d = weight.shape
    b = b0 * b1
    assert b % CHUNK == 0
    n_rows = b // CHUNK
    idx2 = token_ids.astype(jnp.int32).reshape(n_rows, CHUNK)
    out = _gather_rows(weight, idx2, n_rows, d)
    return out.reshape(b0, b1, d)


# speedup vs baseline: 1.0195x; 1.0195x over previous
import functools

import jax
import jax.numpy as jnp
from jax import lax
from jax.experimental import pallas as pl
from jax.experimental.pallas import tpu as pltpu
from jax.experimental.pallas import tpu_sc as plsc

CHUNK = 128
K = 4
NBUF = 2


@functools.partial(jax.jit, static_argnums=(2, 3))
def _gather_rows(weight, idx2, n_rows, d):
    info = plsc.get_sparse_core_info()
    nc, ns = info.num_cores, info.num_subcores
    nw = nc * ns
    rows_pw = n_rows // nw
    n_stages = rows_pw // K
    n_body = n_stages // NBUF - 1

    mesh = plsc.VectorSubcoreMesh(core_axis_name="c", subcore_axis_name="s")

    @functools.partial(
        pl.kernel,
        mesh=mesh,
        compiler_params=pltpu.CompilerParams(use_tc_tiling_on_sc=False),
        out_type=jax.ShapeDtypeStruct((n_rows, CHUNK, d), jnp.float32),
        scratch_types=[
            pltpu.VMEM((rows_pw, CHUNK), jnp.int32),
            pltpu.VMEM((K, CHUNK, d), jnp.float32),
            pltpu.VMEM((K, CHUNK, d), jnp.float32),
            pltpu.SemaphoreType.DMA,
            pltpu.SemaphoreType.DMA,
            pltpu.SemaphoreType.DMA,
            pltpu.SemaphoreType.DMA,
        ],
    )
    def k(table_hbm, idx_hbm, out_hbm, idx_v, rows0, rows1, g0, g1, s0, s1):
        wid = lax.axis_index("s") * nc + lax.axis_index("c")
        base = wid * rows_pw

        pltpu.sync_copy(idx_hbm.at[pl.ds(base, rows_pw)], idx_v)

        def fire_gather(stage, rows_v, sem):
            for j in range(K):
                pltpu.async_copy(
                    table_hbm.at[idx_v.at[stage * K + j]], rows_v.at[j], sem
                )

        def drain_gather(rows_v, sem):
            for j in range(K):
                pltpu.make_async_copy(
                    table_hbm.at[idx_v.at[j]], rows_v.at[j], sem
                ).wait()

        def drain_store(rows_v, sem):
            pltpu.make_async_copy(
                rows_v, out_hbm.at[pl.ds(base, K)], sem
            ).wait()

        def fire_store(stage, rows_v, sem):
            pltpu.async_copy(rows_v, out_hbm.at[pl.ds(base + stage * K, K)], sem)

        fire_gather(0, rows0, g0)
        fire_gather(1, rows1, g1)

        def body(t, carry):
            st0 = NBUF * t
            drain_gather(rows0, g0)
            fire_store(st0, rows0, s0)
            drain_store(rows0, s0)
            fire_gather(st0 + NBUF, rows0, g0)
            drain_gather(rows1, g1)
            fire_store(st0 + 1, rows1, s1)
            drain_store(rows1, s1)
            fire_gather(st0 + 1 + NBUF, rows1, g1)
            return carry

        lax.fori_loop(0, n_body, body, 0)

        last0 = NBUF * n_body
        drain_gather(rows0, g0)
        fire_store(last0, rows0, s0)
        drain_gather(rows1, g1)
        fire_store(last0 + 1, rows1, s1)
        drain_store(rows0, s0)
        drain_store(rows1, s1)

    return k(weight, idx2)


def kernel(token_ids, weight):
    b0, b1 = token_ids.shape
    v, d = weight.shape
    b = b0 * b1
    assert b % CHUNK == 0
    n_rows = b // CHUNK
    idx2 = token_ids.astype(jnp.int32).reshape(n_rows, CHUNK)
    out = _gather_rows(weight, idx2, n_rows, d)
    return out.reshape(b0, b1, d)
